# baseline (speedup 1.0000x reference)
import functools

import jax
import jax.numpy as jnp
from jax import lax
from jax.experimental import pallas as pl
from jax.experimental.pallas import tpu as pltpu
from jax.experimental.pallas import tpu_sc as plsc

VOCAB = 256
EMBED = 64
B = 4096
L = 200

NC = 2
NS = 16
NW = NC * NS

ROWS_PER_W = B // NW
TOK_PER_W = ROWS_PER_W * L
CHUNK = 128
NCHUNK = TOK_PER_W // CHUNK

_MESH = plsc.VectorSubcoreMesh(core_axis_name="c", subcore_axis_name="s")


@functools.partial(
    pl.kernel,
    out_type=jax.ShapeDtypeStruct((B * L, EMBED), jnp.float32),
    mesh=_MESH,
    compiler_params=pltpu.CompilerParams(
        use_tc_tiling_on_sc=False, needs_layout_passes=False
    ),
    scratch_types=[
        pltpu.VMEM((ROWS_PER_W,), jnp.int32),
        pltpu.VMEM((CHUNK,), jnp.int32),
        pltpu.VMEM((CHUNK, EMBED), jnp.float32),
        pltpu.SemaphoreType.DMA,
    ],
)
def _emb_kernel(table_hbm, idx_hbm, len_hbm, out_hbm, len_v, idx_v, rows_v, sem):
    wid = lax.axis_index("c") * NS + lax.axis_index("s")
    row_base = wid * ROWS_PER_W
    tok_base = wid * TOK_PER_W

    pltpu.sync_copy(len_hbm.at[pl.ds(row_base, ROWS_PER_W)], len_v)

    iota = lax.iota(jnp.int32, 16)

    @pl.loop(0, NCHUNK)
    def body(c):
        tok0 = tok_base + c * CHUNK
        pltpu.sync_copy(idx_hbm.at[pl.ds(tok0, CHUNK)], idx_v)
        for v in range(CHUNK // 16):
            t_local = c * CHUNK + v * 16 + iota
            row_local = t_local // L
            pos = t_local - row_local * L
            lens = plsc.load_gather(len_v, [row_local])
            raw = idx_v[pl.ds(v * 16, 16)]
            idx_v[pl.ds(v * 16, 16)] = jnp.where(pos < lens, raw, VOCAB)
        pltpu.async_copy(table_hbm.at[idx_v], rows_v, sem).wait()
        pltpu.sync_copy(rows_v, out_hbm.at[pl.ds(tok0, CHUNK)])


def kernel(vectorized_seqs, seq_lengths, weight):
    idx_flat = vectorized_seqs.reshape(B * L)
    table_pad = jnp.concatenate(
        [weight, jnp.zeros((8, EMBED), jnp.float32)], axis=0
    )
    out = _emb_kernel(table_pad, idx_flat, seq_lengths)
    return out.reshape(B, L, EMBED)
